# baseline (speedup 1.0000x reference)
import functools

import jax
import jax.numpy as jnp
from jax import lax
from jax.experimental import pallas as pl
from jax.experimental.pallas import tpu as pltpu
from jax.experimental.pallas import tpu_sc as plsc

_VOCAB = 26
_HID = 640
_PAD = 1
_MASK = 24
_SPAD = 1024
_MAXPOS = 1026
_EPS = 1e-12
_MASK_RATIO_TRAIN = 0.15 * 0.8

_NC = 2
_NS = 16
_NW = _NC * _NS
_CH = 64

_TBLK = 512


def _stage1_body(ids_ref, pos_ref, a_ref):
    ids = ids_ref[...]
    maskf = (ids != _PAD).astype(jnp.float32)
    s = ids.shape[1]
    row = lax.broadcasted_iota(jnp.int32, (s, s), 0)
    col = lax.broadcasted_iota(jnp.int32, (s, s), 1)
    tri = (row <= col).astype(jnp.bfloat16)
    inc = jnp.dot(maskf.astype(jnp.bfloat16), tri,
                  preferred_element_type=jnp.float32)
    pos_ref[...] = (inc * maskf + 2.0).astype(jnp.int32)
    ismask = ids == _MASK
    nmask = jnp.sum(ismask.astype(jnp.float32), axis=1, keepdims=True)
    srclen = jnp.sum(maskf, axis=1, keepdims=True)
    scale = (1.0 - _MASK_RATIO_TRAIN) / (1.0 - nmask / srclen)
    a_ref[...] = jnp.where(ismask, 0.0, scale)


def _main_body(ids_ref, a_ref, w_ref, lnw_ref, lnb_ref, pg_ref, out_ref):
    ids = ids_ref[0]
    af = a_ref[0]
    voc = lax.broadcasted_iota(jnp.int32, (_VOCAB, 1), 0)
    oht = jnp.where(ids == voc, af, 0.0)
    wg = lax.dot_general(oht, w_ref[...], (((0,), (0,)), ((), ())),
                         precision=lax.Precision.HIGHEST,
                         preferred_element_type=jnp.float32)
    e = (wg[:, _HID:] == 0.0).astype(jnp.float32)
    x = wg[:, :_HID] + pg_ref[...]
    mu = jnp.mean(x, axis=1, keepdims=True)
    xc = x - mu
    var = jnp.mean(xc * xc, axis=1, keepdims=True)
    y = xc * lax.rsqrt(var + _EPS) * lnw_ref[...] + lnb_ref[...]
    out_ref[...] = (y * e)[None]


def _sc_gather(table, idx):
    n = idx.shape[0]
    width = table.shape[1]
    b_per_w = n // _NW
    mesh = plsc.VectorSubcoreMesh(core_axis_name="c", subcore_axis_name="s")

    nch = b_per_w // _CH

    @functools.partial(
        pl.kernel,
        mesh=mesh,
        out_type=jax.ShapeDtypeStruct((n, width), table.dtype),
        scratch_types=[
            pltpu.VMEM((b_per_w,), jnp.int32),
            pltpu.VMEM((_CH, width), table.dtype),
            pltpu.VMEM((_CH, width), table.dtype),
            pltpu.SemaphoreType.DMA,
            pltpu.SemaphoreType.DMA,
            pltpu.SemaphoreType.DMA,
            pltpu.SemaphoreType.DMA,
        ],
    )
    def k(table_hbm, idx_hbm, out_hbm, idx_v, buf0, buf1, g0, g1, w0, w1):
        wid = lax.axis_index("s") * _NC + lax.axis_index("c")
        base = wid * b_per_w
        pltpu.sync_copy(idx_hbm.at[pl.ds(base, b_per_w)], idx_v)

        def start_g(ci, buf, sem):
            pltpu.async_copy(table_hbm.at[idx_v.at[pl.ds(ci * _CH, _CH)]],
                             buf, sem)

        def wait_g(ci, buf, sem):
            pltpu.make_async_copy(table_hbm.at[idx_v.at[pl.ds(ci * _CH, _CH)]],
                                  buf, sem).wait()

        def start_w(ci, buf, sem):
            pltpu.async_copy(buf, out_hbm.at[pl.ds(base + ci * _CH, _CH)], sem)

        def wait_w(ci, buf, sem):
            pltpu.make_async_copy(buf, out_hbm.at[pl.ds(base + ci * _CH, _CH)],
                                  sem).wait()

        start_g(0, buf0, g0)
        start_g(1, buf1, g1)

        @pl.loop(0, nch, step=2)
        def _(ci):
            wait_g(ci, buf0, g0)
            start_w(ci, buf0, w0)
            wait_g(ci + 1, buf1, g1)
            wait_w(ci, buf0, w0)

            @pl.when(ci + 2 < nch)
            def _():
                start_g(ci + 2, buf0, g0)

            start_w(ci + 1, buf1, w1)
            wait_w(ci + 1, buf1, w1)

            @pl.when(ci + 3 < nch)
            def _():
                start_g(ci + 3, buf1, g1)

    return k(table, idx)


def kernel(input_ids, word_emb, pos_emb, ln_w, ln_b):
    ids = input_ids.astype(jnp.int32)
    b, s = ids.shape
    ids_p = jnp.pad(ids, ((0, 0), (0, _SPAD - s)), constant_values=_PAD)
    n = b * _SPAD

    pos, a = pl.pallas_call(
        _stage1_body,
        out_shape=[
            jax.ShapeDtypeStruct((b, _SPAD), jnp.int32),
            jax.ShapeDtypeStruct((b, _SPAD), jnp.float32),
        ],
    )(ids_p)

    pg = _sc_gather(pos_emb, pos.reshape(n))

    w_aug = jnp.concatenate(
        [word_emb,
         (jnp.arange(_VOCAB) == _PAD).astype(jnp.float32)[:, None]], axis=1)

    nj = _SPAD // _TBLK
    out = pl.pallas_call(
        _main_body,
        grid=(b, nj),
        in_specs=[
            pl.BlockSpec((1, 1, _TBLK), lambda i, j: (i, 0, j)),
            pl.BlockSpec((1, 1, _TBLK), lambda i, j: (i, 0, j)),
            pl.BlockSpec((_VOCAB, _HID + 1), lambda i, j: (0, 0)),
            pl.BlockSpec((1, _HID), lambda i, j: (0, 0)),
            pl.BlockSpec((1, _HID), lambda i, j: (0, 0)),
            pl.BlockSpec((_TBLK, _HID), lambda i, j: (i * nj + j, 0)),
        ],
        out_specs=pl.BlockSpec((1, _TBLK, _HID), lambda i, j: (i, j, 0)),
        out_shape=jax.ShapeDtypeStruct((b, s, _HID), jnp.float32),
    )(
        ids_p.reshape(b, 1, _SPAD),
        a.reshape(b, 1, _SPAD),
        w_aug,
        ln_w.reshape(1, _HID),
        ln_b.reshape(1, _HID),
        pg,
    )
    return out


# speedup vs baseline: 1.1835x; 1.1835x over previous
import functools

import jax
import jax.numpy as jnp
from jax import lax
from jax.experimental import pallas as pl
from jax.experimental.pallas import tpu as pltpu
from jax.experimental.pallas import tpu_sc as plsc

_VOCAB = 26
_HID = 640
_PAD = 1
_MASK = 24
_SPAD = 1024
_MAXPOS = 1026
_EPS = 1e-12
_MASK_RATIO_TRAIN = 0.15 * 0.8

_NC = 2
_NS = 16
_NW = _NC * _NS
_CH = 64

_TBLK = 512


def _stage1_body(ids_ref, pos_ref, a_ref):
    ids = ids_ref[...]
    maskf = (ids != _PAD).astype(jnp.float32)
    s = ids.shape[1]
    row = lax.broadcasted_iota(jnp.int32, (s, s), 0)
    col = lax.broadcasted_iota(jnp.int32, (s, s), 1)
    tri = (row <= col).astype(jnp.bfloat16)
    inc = jnp.dot(maskf.astype(jnp.bfloat16), tri,
                  preferred_element_type=jnp.float32)
    pos_ref[...] = (inc * maskf + 2.0).astype(jnp.int32)
    ismask = ids == _MASK
    nmask = jnp.sum(ismask.astype(jnp.float32), axis=1, keepdims=True)
    srclen = jnp.sum(maskf, axis=1, keepdims=True)
    scale = (1.0 - _MASK_RATIO_TRAIN) / (1.0 - nmask / srclen)
    a_ref[...] = jnp.where(ismask, 0.0, scale)


def _main_body(ids_ref, a_ref, w_ref, lnw_ref, lnb_ref, pg_ref, out_ref):
    ids = ids_ref[0]
    af = a_ref[0]
    voc = lax.broadcasted_iota(jnp.int32, (_VOCAB, 1), 0)
    oht = jnp.where(ids == voc, af, 0.0)
    wg = lax.dot_general(oht, w_ref[...], (((0,), (0,)), ((), ())),
                         precision=lax.Precision.HIGHEST,
                         preferred_element_type=jnp.float32)
    e = (wg[:, _HID:] == 0.0).astype(jnp.float32)
    x = wg[:, :_HID] + pg_ref[...]
    mu = jnp.mean(x, axis=1, keepdims=True)
    xc = x - mu
    var = jnp.mean(xc * xc, axis=1, keepdims=True)
    y = xc * lax.rsqrt(var + _EPS) * lnw_ref[...] + lnb_ref[...]
    out_ref[...] = (y * e)[None]


def _sc_gather(table, idx):
    n = idx.shape[0]
    width = table.shape[1]
    b_per_w = n // _NW
    mesh = plsc.VectorSubcoreMesh(core_axis_name="c", subcore_axis_name="s")

    nch = b_per_w // _CH

    @functools.partial(
        pl.kernel,
        mesh=mesh,
        out_type=jax.ShapeDtypeStruct((n, width), table.dtype),
        scratch_types=[
            pltpu.VMEM((b_per_w,), jnp.int32),
            pltpu.VMEM((_CH, width), table.dtype),
            pltpu.VMEM((_CH, width), table.dtype),
            pltpu.SemaphoreType.DMA,
            pltpu.SemaphoreType.DMA,
            pltpu.SemaphoreType.DMA,
            pltpu.SemaphoreType.DMA,
        ],
    )
    def k(table_hbm, idx_hbm, out_hbm, idx_v, buf0, buf1, g0, g1, w0, w1):
        wid = lax.axis_index("s") * _NC + lax.axis_index("c")
        base = wid * b_per_w
        pltpu.sync_copy(idx_hbm.at[pl.ds(base, b_per_w)], idx_v)

        def start_g(ci, buf, sem):
            pltpu.async_copy(table_hbm.at[idx_v.at[pl.ds(ci * _CH, _CH)]],
                             buf, sem)

        def wait_g(ci, buf, sem):
            pltpu.make_async_copy(table_hbm.at[idx_v.at[pl.ds(ci * _CH, _CH)]],
                                  buf, sem).wait()

        def start_w(ci, buf, sem):
            pltpu.async_copy(buf, out_hbm.at[pl.ds(base + ci * _CH, _CH)], sem)

        def wait_w(ci, buf, sem):
            pltpu.make_async_copy(buf, out_hbm.at[pl.ds(base + ci * _CH, _CH)],
                                  sem).wait()

        start_g(0, buf0, g0)
        start_g(1, buf1, g1)

        @pl.loop(0, nch, step=2)
        def _(ci):
            wait_g(ci, buf0, g0)
            start_w(ci, buf0, w0)
            wait_g(ci + 1, buf1, g1)
            wait_w(ci, buf0, w0)

            @pl.when(ci + 2 < nch)
            def _():
                start_g(ci + 2, buf0, g0)

            start_w(ci + 1, buf1, w1)
            wait_w(ci + 1, buf1, w1)

            @pl.when(ci + 3 < nch)
            def _():
                start_g(ci + 3, buf1, g1)

    return k(table, idx)


def _main_body_acc(prev_ref, ids_ref, a_ref, w_ref, lnw_ref, lnb_ref, pg_ref,
                   out_ref):
    del prev_ref
    _main_body(ids_ref, a_ref, w_ref, lnw_ref, lnb_ref, pg_ref, out_ref)


_NCHUNK = 4


def kernel(input_ids, word_emb, pos_emb, ln_w, ln_b):
    ids = input_ids.astype(jnp.int32)
    b, s = ids.shape
    ids_p = jnp.pad(ids, ((0, 0), (0, _SPAD - s)), constant_values=_PAD)
    n = b * _SPAD
    bc = b // _NCHUNK

    pos, a = pl.pallas_call(
        _stage1_body,
        out_shape=[
            jax.ShapeDtypeStruct((b, _SPAD), jnp.int32),
            jax.ShapeDtypeStruct((b, _SPAD), jnp.float32),
        ],
    )(ids_p)

    w_aug = jnp.concatenate(
        [word_emb,
         (jnp.arange(_VOCAB) == _PAD).astype(jnp.float32)[:, None]], axis=1)

    nj = _SPAD // _TBLK
    posf = pos.reshape(_NCHUNK, n // _NCHUNK)
    ids3 = ids_p.reshape(b, 1, _SPAD)
    a3 = a.reshape(b, 1, _SPAD)

    out = None
    for k in range(_NCHUNK):
        pg_k = _sc_gather(pos_emb, posf[k])

        def mk_spec(k):
            return [
                pl.BlockSpec((1, 1, _TBLK), lambda i, j: (bc * k + i, 0, j)),
                pl.BlockSpec((1, 1, _TBLK), lambda i, j: (bc * k + i, 0, j)),
                pl.BlockSpec((_VOCAB, _HID + 1), lambda i, j: (0, 0)),
                pl.BlockSpec((1, _HID), lambda i, j: (0, 0)),
                pl.BlockSpec((1, _HID), lambda i, j: (0, 0)),
                pl.BlockSpec((_TBLK, _HID), lambda i, j: (i * nj + j, 0)),
            ]

        out_spec = pl.BlockSpec((1, _TBLK, _HID),
                                functools.partial(
                                    lambda k, i, j: (bc * k + i, j, 0), k))
        out_shape = jax.ShapeDtypeStruct((b, s, _HID), jnp.float32)
        if out is None:
            out = pl.pallas_call(
                _main_body,
                grid=(bc, nj),
                in_specs=mk_spec(k),
                out_specs=out_spec,
                out_shape=out_shape,
            )(ids3, a3, w_aug, ln_w.reshape(1, _HID), ln_b.reshape(1, _HID),
              pg_k)
        else:
            out = pl.pallas_call(
                _main_body_acc,
                grid=(bc, nj),
                in_specs=[pl.BlockSpec(memory_space=pltpu.MemorySpace.HBM)]
                + mk_spec(k),
                out_specs=out_spec,
                out_shape=out_shape,
                input_output_aliases={0: 0},
            )(out, ids3, a3, w_aug, ln_w.reshape(1, _HID),
              ln_b.reshape(1, _HID), pg_k)
    return out


# speedup vs baseline: 1.2197x; 1.0306x over previous
import functools

import jax
import jax.numpy as jnp
from jax import lax
from jax.experimental import pallas as pl
from jax.experimental.pallas import tpu as pltpu
from jax.experimental.pallas import tpu_sc as plsc

_VOCAB = 26
_HID = 640
_PAD = 1
_MASK = 24
_SPAD = 1024
_MAXPOS = 1026
_EPS = 1e-12
_MASK_RATIO_TRAIN = 0.15 * 0.8

_NC = 2
_NS = 16
_NW = _NC * _NS
_CH = 64

_TBLK = 1024


def _stage1_body(ids_ref, pos_ref, a_ref):
    ids = ids_ref[...]
    maskf = (ids != _PAD).astype(jnp.float32)
    s = ids.shape[1]
    row = lax.broadcasted_iota(jnp.int32, (s, s), 0)
    col = lax.broadcasted_iota(jnp.int32, (s, s), 1)
    tri = (row <= col).astype(jnp.bfloat16)
    inc = jnp.dot(maskf.astype(jnp.bfloat16), tri,
                  preferred_element_type=jnp.float32)
    pos_ref[...] = (inc * maskf + 2.0).astype(jnp.int32)
    ismask = ids == _MASK
    nmask = jnp.sum(ismask.astype(jnp.float32), axis=1, keepdims=True)
    srclen = jnp.sum(maskf, axis=1, keepdims=True)
    scale = (1.0 - _MASK_RATIO_TRAIN) / (1.0 - nmask / srclen)
    a_ref[...] = jnp.where(ismask, 0.0, scale)


def _main_body(ids_ref, a_ref, w_ref, lnw_ref, lnb_ref, pg_ref, out_ref):
    ids = ids_ref[0]
    af = a_ref[0]
    voc = lax.broadcasted_iota(jnp.int32, (_VOCAB, 1), 0)
    oht = jnp.where(ids == voc, af, 0.0)
    wg = lax.dot_general(oht, w_ref[...], (((0,), (0,)), ((), ())),
                         precision=lax.Precision.HIGHEST,
                         preferred_element_type=jnp.float32)
    e = (wg[:, _HID:] == 0.0).astype(jnp.float32)
    x = wg[:, :_HID] + pg_ref[...]
    mu = jnp.mean(x, axis=1, keepdims=True)
    xc = x - mu
    var = jnp.mean(xc * xc, axis=1, keepdims=True)
    y = xc * lax.rsqrt(var + _EPS) * lnw_ref[...] + lnb_ref[...]
    out_ref[...] = (y * e)[None]


def _sc_gather(table, idx):
    n = idx.shape[0]
    width = table.shape[1]
    b_per_w = n // _NW
    mesh = plsc.VectorSubcoreMesh(core_axis_name="c", subcore_axis_name="s")

    nch = b_per_w // _CH

    @functools.partial(
        pl.kernel,
        mesh=mesh,
        out_type=jax.ShapeDtypeStruct((n, width), table.dtype),
        scratch_types=[
            pltpu.VMEM((b_per_w,), jnp.int32),
            pltpu.VMEM((_CH, width), table.dtype),
            pltpu.VMEM((_CH, width), table.dtype),
            pltpu.SemaphoreType.DMA,
            pltpu.SemaphoreType.DMA,
            pltpu.SemaphoreType.DMA,
            pltpu.SemaphoreType.DMA,
        ],
    )
    def k(table_hbm, idx_hbm, out_hbm, idx_v, buf0, buf1, g0, g1, w0, w1):
        wid = lax.axis_index("s") * _NC + lax.axis_index("c")
        base = wid * b_per_w
        pltpu.sync_copy(idx_hbm.at[pl.ds(base, b_per_w)], idx_v)

        def start_g(ci, buf, sem):
            pltpu.async_copy(table_hbm.at[idx_v.at[pl.ds(ci * _CH, _CH)]],
                             buf, sem)

        def wait_g(ci, buf, sem):
            pltpu.make_async_copy(table_hbm.at[idx_v.at[pl.ds(ci * _CH, _CH)]],
                                  buf, sem).wait()

        def start_w(ci, buf, sem):
            pltpu.async_copy(buf, out_hbm.at[pl.ds(base + ci * _CH, _CH)], sem)

        def wait_w(ci, buf, sem):
            pltpu.make_async_copy(buf, out_hbm.at[pl.ds(base + ci * _CH, _CH)],
                                  sem).wait()

        start_g(0, buf0, g0)
        start_g(1, buf1, g1)

        @pl.loop(0, nch, step=2)
        def _(ci):
            wait_g(ci, buf0, g0)
            start_w(ci, buf0, w0)
            wait_g(ci + 1, buf1, g1)
            wait_w(ci, buf0, w0)

            @pl.when(ci + 2 < nch)
            def _():
                start_g(ci + 2, buf0, g0)

            start_w(ci + 1, buf1, w1)
            wait_w(ci + 1, buf1, w1)

            @pl.when(ci + 3 < nch)
            def _():
                start_g(ci + 3, buf1, g1)

    return k(table, idx)


def _main_body_acc(prev_ref, ids_ref, a_ref, w_ref, lnw_ref, lnb_ref, pg_ref,
                   out_ref):
    del prev_ref
    _main_body(ids_ref, a_ref, w_ref, lnw_ref, lnb_ref, pg_ref, out_ref)


_NCHUNK = 4


def kernel(input_ids, word_emb, pos_emb, ln_w, ln_b):
    ids = input_ids.astype(jnp.int32)
    b, s = ids.shape
    ids_p = jnp.pad(ids, ((0, 0), (0, _SPAD - s)), constant_values=_PAD)
    n = b * _SPAD
    bc = b // _NCHUNK

    pos, a = pl.pallas_call(
        _stage1_body,
        out_shape=[
            jax.ShapeDtypeStruct((b, _SPAD), jnp.int32),
            jax.ShapeDtypeStruct((b, _SPAD), jnp.float32),
        ],
    )(ids_p)

    w_aug = jnp.concatenate(
        [word_emb,
         (jnp.arange(_VOCAB) == _PAD).astype(jnp.float32)[:, None]], axis=1)

    nj = _SPAD // _TBLK
    posf = pos.reshape(_NCHUNK, n // _NCHUNK)
    ids3 = ids_p.reshape(b, 1, _SPAD)
    a3 = a.reshape(b, 1, _SPAD)

    out = None
    for k in range(_NCHUNK):
        pg_k = _sc_gather(pos_emb, posf[k])

        def mk_spec(k):
            return [
                pl.BlockSpec((1, 1, _TBLK), lambda i, j: (bc * k + i, 0, j)),
                pl.BlockSpec((1, 1, _TBLK), lambda i, j: (bc * k + i, 0, j)),
                pl.BlockSpec((_VOCAB, _HID + 1), lambda i, j: (0, 0)),
                pl.BlockSpec((1, _HID), lambda i, j: (0, 0)),
                pl.BlockSpec((1, _HID), lambda i, j: (0, 0)),
                pl.BlockSpec((_TBLK, _HID), lambda i, j: (i * nj + j, 0)),
            ]

        out_spec = pl.BlockSpec((1, _TBLK, _HID),
                                functools.partial(
                                    lambda k, i, j: (bc * k + i, j, 0), k))
        out_shape = jax.ShapeDtypeStruct((b, s, _HID), jnp.float32)
        if out is None:
            out = pl.pallas_call(
                _main_body,
                grid=(bc, nj),
                in_specs=mk_spec(k),
                out_specs=out_spec,
                out_shape=out_shape,
            )(ids3, a3, w_aug, ln_w.reshape(1, _HID), ln_b.reshape(1, _HID),
              pg_k)
        else:
            out = pl.pallas_call(
                _main_body_acc,
                grid=(bc, nj),
                in_specs=[pl.BlockSpec(memory_space=pltpu.MemorySpace.HBM)]
                + mk_spec(k),
                out_specs=out_spec,
                out_shape=out_shape,
                input_output_aliases={0: 0},
            )(out, ids3, a3, w_aug, ln_w.reshape(1, _HID),
              ln_b.reshape(1, _HID), pg_k)
    return out


# speedup vs baseline: 1.2413x; 1.0177x over previous
import functools

import jax
import jax.numpy as jnp
from jax import lax
from jax.experimental import pallas as pl
from jax.experimental.pallas import tpu as pltpu
from jax.experimental.pallas import tpu_sc as plsc

_VOCAB = 26
_HID = 640
_PAD = 1
_MASK = 24
_SPAD = 1024
_MAXPOS = 1026
_EPS = 1e-12
_MASK_RATIO_TRAIN = 0.15 * 0.8

_NC = 2
_NS = 16
_NW = _NC * _NS
_CH = 64

_TBLK = 1024


def _stage1_body(ids_ref, pos_ref, a_ref):
    ids = ids_ref[...]
    maskf = (ids != _PAD).astype(jnp.float32)
    s = ids.shape[1]
    row = lax.broadcasted_iota(jnp.int32, (s, s), 0)
    col = lax.broadcasted_iota(jnp.int32, (s, s), 1)
    tri = (row <= col).astype(jnp.bfloat16)
    inc = jnp.dot(maskf.astype(jnp.bfloat16), tri,
                  preferred_element_type=jnp.float32)
    pos_ref[...] = (inc * maskf + 2.0).astype(jnp.int32)
    ismask = ids == _MASK
    nmask = jnp.sum(ismask.astype(jnp.float32), axis=1, keepdims=True)
    srclen = jnp.sum(maskf, axis=1, keepdims=True)
    scale = (1.0 - _MASK_RATIO_TRAIN) / (1.0 - nmask / srclen)
    a_ref[...] = jnp.where(ismask, 0.0, scale)


def _main_body(ids_ref, a_ref, w_ref, lnw_ref, lnb_ref, pg_ref, out_ref):
    ids = ids_ref[0]
    af = a_ref[0]
    voc = lax.broadcasted_iota(jnp.int32, (_VOCAB, 1), 0)
    oht = jnp.where(ids == voc, af, 0.0)
    wg = lax.dot_general(oht, w_ref[...], (((0,), (0,)), ((), ())),
                         precision=lax.Precision.HIGHEST,
                         preferred_element_type=jnp.float32)
    e = (wg[:, _HID:] == 0.0).astype(jnp.float32)
    x = wg[:, :_HID] + pg_ref[...]
    mu = jnp.mean(x, axis=1, keepdims=True)
    xc = x - mu
    var = jnp.mean(xc * xc, axis=1, keepdims=True)
    y = xc * lax.rsqrt(var + _EPS) * lnw_ref[...] + lnb_ref[...]
    out_ref[...] = (y * e)[None]


def _sc_gather(table, idx):
    n = idx.shape[0]
    width = table.shape[1]
    b_per_w = n // _NW
    mesh = plsc.VectorSubcoreMesh(core_axis_name="c", subcore_axis_name="s")

    nch = b_per_w // _CH

    @functools.partial(
        pl.kernel,
        mesh=mesh,
        out_type=jax.ShapeDtypeStruct((n, width), table.dtype),
        scratch_types=[
            pltpu.VMEM((b_per_w,), jnp.int32),
            pltpu.VMEM((_CH, width), table.dtype),
            pltpu.VMEM((_CH, width), table.dtype),
            pltpu.SemaphoreType.DMA,
            pltpu.SemaphoreType.DMA,
            pltpu.SemaphoreType.DMA,
            pltpu.SemaphoreType.DMA,
        ],
    )
    def k(table_hbm, idx_hbm, out_hbm, idx_v, buf0, buf1, g0, g1, w0, w1):
        wid = lax.axis_index("s") * _NC + lax.axis_index("c")
        base = wid * b_per_w
        pltpu.sync_copy(idx_hbm.at[pl.ds(base, b_per_w)], idx_v)

        def start_g(ci, buf, sem):
            pltpu.async_copy(table_hbm.at[idx_v.at[pl.ds(ci * _CH, _CH)]],
                             buf, sem)

        def wait_g(ci, buf, sem):
            pltpu.make_async_copy(table_hbm.at[idx_v.at[pl.ds(ci * _CH, _CH)]],
                                  buf, sem).wait()

        def start_w(ci, buf, sem):
            pltpu.async_copy(buf, out_hbm.at[pl.ds(base + ci * _CH, _CH)], sem)

        def wait_w(ci, buf, sem):
            pltpu.make_async_copy(buf, out_hbm.at[pl.ds(base + ci * _CH, _CH)],
                                  sem).wait()

        start_g(0, buf0, g0)
        start_g(1, buf1, g1)

        @pl.loop(0, nch, step=2)
        def _(ci):
            wait_g(ci, buf0, g0)
            start_w(ci, buf0, w0)
            wait_g(ci + 1, buf1, g1)
            wait_w(ci, buf0, w0)

            @pl.when(ci + 2 < nch)
            def _():
                start_g(ci + 2, buf0, g0)

            start_w(ci + 1, buf1, w1)
            wait_w(ci + 1, buf1, w1)

            @pl.when(ci + 3 < nch)
            def _():
                start_g(ci + 3, buf1, g1)

    return k(table, idx)


def _main_body_acc(prev_ref, ids_ref, a_ref, w_ref, lnw_ref, lnb_ref, pg_ref,
                   out_ref):
    del prev_ref
    _main_body(ids_ref, a_ref, w_ref, lnw_ref, lnb_ref, pg_ref, out_ref)


_NCHUNK = 8


def kernel(input_ids, word_emb, pos_emb, ln_w, ln_b):
    ids = input_ids.astype(jnp.int32)
    b, s = ids.shape
    ids_p = jnp.pad(ids, ((0, 0), (0, _SPAD - s)), constant_values=_PAD)
    n = b * _SPAD
    bc = b // _NCHUNK

    pos, a = pl.pallas_call(
        _stage1_body,
        out_shape=[
            jax.ShapeDtypeStruct((b, _SPAD), jnp.int32),
            jax.ShapeDtypeStruct((b, _SPAD), jnp.float32),
        ],
    )(ids_p)

    w_aug = jnp.concatenate(
        [word_emb,
         (jnp.arange(_VOCAB) == _PAD).astype(jnp.float32)[:, None]], axis=1)

    nj = _SPAD // _TBLK
    posf = pos.reshape(_NCHUNK, n // _NCHUNK)
    ids3 = ids_p.reshape(b, 1, _SPAD)
    a3 = a.reshape(b, 1, _SPAD)

    out = None
    for k in range(_NCHUNK):
        pg_k = _sc_gather(pos_emb, posf[k])

        def mk_spec(k):
            return [
                pl.BlockSpec((1, 1, _TBLK), lambda i, j: (bc * k + i, 0, j)),
                pl.BlockSpec((1, 1, _TBLK), lambda i, j: (bc * k + i, 0, j)),
                pl.BlockSpec((_VOCAB, _HID + 1), lambda i, j: (0, 0)),
                pl.BlockSpec((1, _HID), lambda i, j: (0, 0)),
                pl.BlockSpec((1, _HID), lambda i, j: (0, 0)),
                pl.BlockSpec((_TBLK, _HID), lambda i, j: (i * nj + j, 0)),
            ]

        out_spec = pl.BlockSpec((1, _TBLK, _HID),
                                functools.partial(
                                    lambda k, i, j: (bc * k + i, j, 0), k))
        out_shape = jax.ShapeDtypeStruct((b, s, _HID), jnp.float32)
        if out is None:
            out = pl.pallas_call(
                _main_body,
                grid=(bc, nj),
                in_specs=mk_spec(k),
                out_specs=out_spec,
                out_shape=out_shape,
            )(ids3, a3, w_aug, ln_w.reshape(1, _HID), ln_b.reshape(1, _HID),
              pg_k)
        else:
            out = pl.pallas_call(
                _main_body_acc,
                grid=(bc, nj),
                in_specs=[pl.BlockSpec(memory_space=pltpu.MemorySpace.HBM)]
                + mk_spec(k),
                out_specs=out_spec,
                out_shape=out_shape,
                input_output_aliases={0: 0},
            )(out, ids3, a3, w_aug, ln_w.reshape(1, _HID),
              ln_b.reshape(1, _HID), pg_k)
    return out
